# baseline (speedup 1.0000x reference)
import functools

import jax
import jax.numpy as jnp
from jax import lax
from jax.experimental import pallas as pl
from jax.experimental.pallas import tpu as pltpu
from jax.experimental.pallas import tpu_sc as plsc

_N = 10000
_E = 320000
_NPAD = 10240
_NC, _NS = 2, 16
_NW = _NC * _NS
_RPT = _NPAD // _NS


def _epad(K):
    return ((_E + 2 * _NW * K - 1) // (2 * _NW * K)) * (2 * _NW * K)


def _make_sc_edge(C, K):
    mesh = plsc.VectorSubcoreMesh(core_axis_name="c", subcore_axis_name="s")
    T = _epad(K) // _NW
    NCH = T // K

    @functools.partial(
        pl.kernel,
        out_type=[
            jax.ShapeDtypeStruct((_NC, _NPAD, C), jnp.float32),
            jax.ShapeDtypeStruct((_NC, _NPAD), jnp.float32),
        ],
        mesh=mesh,
        compiler_params=pltpu.CompilerParams(needs_layout_passes=False,
                                             use_tc_tiling_on_sc=False),
        scratch_types=[
            pltpu.VMEM((_NPAD,), jnp.float32),
            pltpu.VMEM((_NPAD,), jnp.float32),
            pltpu.VMEM((16,), jnp.float32),
            pltpu.VMEM((2, 2, K), jnp.int32),
            pltpu.VMEM((2, K, C), jnp.float32),
            pltpu.VMEM((2, K), jnp.float32),
            pltpu.VMEM_SHARED((_NPAD, C), jnp.float32),
            pltpu.VMEM_SHARED((_NPAD,), jnp.float32),
            pltpu.SemaphoreType.DMA,
            pltpu.SemaphoreType.DMA,
            pltpu.SemaphoreType.DMA,
            pltpu.SemaphoreType.DMA,
            pltpu.SemaphoreType.DMA,
            pltpu.SemaphoreType.DMA,
        ],
    )
    def sc_edge(edges_hbm, h_hbm, asrc_hbm, adst_hbm, g_hbm,
                z2_hbm, z1_hbm, acc_out, den_out,
                asrc_l, adst_l, g_l, ebuf, rows, evals, acc_sh, den_sh,
                isem0, isem1, gsem0, gsem1, ssem0, ssem1):
        cid = lax.axis_index("c")
        sid = lax.axis_index("s")
        wid = sid * _NC + cid
        r0 = sid * _RPT
        isem = (isem0, isem1)
        gsem = (gsem0, gsem1)
        ssem = (ssem0, ssem1)

        pltpu.sync_copy(z2_hbm.at[pl.ds(r0, _RPT)], acc_sh.at[pl.ds(r0, _RPT)])
        pltpu.sync_copy(z1_hbm.at[pl.ds(r0, _RPT)], den_sh.at[pl.ds(r0, _RPT)])
        pltpu.sync_copy(asrc_hbm, asrc_l)
        pltpu.sync_copy(adst_hbm, adst_l)
        pltpu.sync_copy(g_hbm, g_l)
        plsc.subcore_barrier()

        g = g_l[...]

        def idx_start(ci, b):
            pltpu.async_copy(edges_hbm.at[wid, ci], ebuf.at[b], isem[b])

        def idx_wait(b):
            pltpu.make_async_copy(edges_hbm.at[wid, 0], ebuf.at[b],
                                  isem[b]).wait()

        def gather_start(b):
            pltpu.async_copy(h_hbm.at[ebuf.at[b, 0]], rows.at[b], gsem[b])

        def gather_wait(b):
            pltpu.make_async_copy(h_hbm.at[pl.ds(0, K)], rows.at[b],
                                  gsem[b]).wait()

        def scatter_start(b):
            pltpu.async_copy(rows.at[b], acc_sh.at[ebuf.at[b, 1]], ssem[b],
                             add=True)
            pltpu.async_copy(evals.at[b], den_sh.at[ebuf.at[b, 1]], ssem[b],
                             add=True)

        def scatter_wait(b):
            pltpu.make_async_copy(z2_hbm.at[pl.ds(0, K)], rows.at[b],
                                  ssem[b]).wait()
            pltpu.make_async_copy(z1_hbm.at[pl.ds(0, K)], evals.at[b],
                                  ssem[b]).wait()

        idx_start(0, 0)
        idx_wait(0)
        gather_start(0)

        def pair(gi, _):
            for b in (0, 1):
                ci = 2 * gi + b
                o = 1 - b

                @pl.when(ci >= 1)
                def _():
                    scatter_wait(o)

                @pl.when(ci + 1 < NCH)
                def _():
                    idx_start(ci + 1, o)
                    idx_wait(o)
                    gather_start(o)

                def ebody(i, _):
                    sv = ebuf[b, 0, pl.ds(i * 16, 16)]
                    dv = ebuf[b, 1, pl.ds(i * 16, 16)]
                    s = (plsc.load_gather(asrc_l, [sv])
                         + plsc.load_gather(adst_l, [dv]))
                    alpha = jnp.where(s >= 0.0, s, 0.2 * s)
                    evals[b, pl.ds(i * 16, 16)] = jnp.exp(alpha - g)
                    return 0

                lax.fori_loop(0, K // 16, ebody, 0)

                gather_wait(b)

                def sbody(i, _):
                    e = evals[b, pl.ds(i * 16, 16)]
                    for jj in range(16):
                        es = e[jj]
                        j = i * 16 + jj
                        for r in range(C // 16):
                            rows[b, j, pl.ds(r * 16, 16)] = (
                                rows[b, j, pl.ds(r * 16, 16)] * es)
                    return 0

                lax.fori_loop(0, K // 16, sbody, 0)

                scatter_start(b)
            return 0

        lax.fori_loop(0, NCH // 2, pair, 0)
        scatter_wait(1)
        plsc.subcore_barrier()

        pltpu.sync_copy(acc_sh.at[pl.ds(r0, _RPT)],
                        acc_out.at[cid, pl.ds(r0, _RPT)])
        pltpu.sync_copy(den_sh.at[pl.ds(r0, _RPT)],
                        den_out.at[cid, pl.ds(r0, _RPT)])

    return sc_edge


_K64, _K128 = 128, 96
_sc_edge64 = _make_sc_edge(64, _K64)
_sc_edge128 = _make_sc_edge(128, _K128)


_R = 1280
_NB = _NPAD // _R


def _g_update(i, s, d, g_ref, ms_ref, md_ref):
    bs = jnp.broadcast_to(jnp.max(s), (1, 1))
    bd = jnp.broadcast_to(jnp.max(d), (1, 1))

    @pl.when(i == 0)
    def _():
        ms_ref[...] = bs
        md_ref[...] = bd

    @pl.when(i > 0)
    def _():
        ms_ref[...] = jnp.maximum(ms_ref[...], bs)
        md_ref[...] = jnp.maximum(md_ref[...], bd)

    @pl.when(i == _NB - 1)
    def _():
        g0 = ms_ref[0, 0] + md_ref[0, 0]
        g_ref[...] = jnp.broadcast_to(jnp.where(g0 >= 0.0, g0, 0.2 * g0),
                                      (1, 1))


def _tc_pre(x_pad, W, a_s, a_d):
    Ci = x_pad.shape[1]
    C = W.shape[1]

    def body(x_ref, w_ref, as_ref, ad_ref,
             h_ref, s_ref, d_ref, g_ref, ms_ref, md_ref):
        i = pl.program_id(0)
        h = jnp.dot(x_ref[...], w_ref[...], preferred_element_type=jnp.float32)
        h_ref[...] = h
        s = jnp.sum(h * as_ref[...], axis=1, keepdims=True)
        d = jnp.sum(h * ad_ref[...], axis=1, keepdims=True)
        s_ref[...] = s
        d_ref[...] = d
        _g_update(i, s, d, g_ref, ms_ref, md_ref)

    out = pl.pallas_call(
        body,
        grid=(_NB,),
        in_specs=[
            pl.BlockSpec((_R, Ci), lambda i: (i, 0)),
            pl.BlockSpec((Ci, C), lambda i: (0, 0)),
            pl.BlockSpec((1, C), lambda i: (0, 0)),
            pl.BlockSpec((1, C), lambda i: (0, 0)),
        ],
        out_specs=[
            pl.BlockSpec((_R, C), lambda i: (i, 0)),
            pl.BlockSpec((_R, 1), lambda i: (i, 0)),
            pl.BlockSpec((_R, 1), lambda i: (i, 0)),
            pl.BlockSpec((1, 1), lambda i: (0, 0)),
            pl.BlockSpec((1, 1), lambda i: (0, 0)),
            pl.BlockSpec((1, 1), lambda i: (0, 0)),
        ],
        out_shape=[
            jax.ShapeDtypeStruct((_NPAD, C), jnp.float32),
            jax.ShapeDtypeStruct((_NPAD, 1), jnp.float32),
            jax.ShapeDtypeStruct((_NPAD, 1), jnp.float32),
            jax.ShapeDtypeStruct((1, 1), jnp.float32),
            jax.ShapeDtypeStruct((1, 1), jnp.float32),
            jax.ShapeDtypeStruct((1, 1), jnp.float32),
        ],
    )(x_pad, W, a_s, a_d)
    return out[0], out[1], out[2], out[3]


def _tc_mid(acc, den, b, W, a_s, a_d, hp, sp, dp, gp):
    C = W.shape[1]

    Cp = hp.shape[1]

    def body(acc_ref, den_ref, b_ref, w_ref, as_ref, ad_ref,
             hp_ref, sp_ref, dp_ref, gp_ref,
             h_ref, s_ref, d_ref, g_ref, ms_ref, md_ref):
        i = pl.program_id(0)
        sd = sp_ref[...] + dp_ref[...]
        esel = jnp.exp(jnp.where(sd >= 0.0, sd, 0.2 * sd) - gp_ref[...])
        a = acc_ref[0] + acc_ref[1] + esel * hp_ref[...]
        dn = den_ref[0] + den_ref[1] + esel
        o = jnp.where(dn > 0.0, a / dn, 0.0) + b_ref[...]
        o = jnp.maximum(o, 0.0)
        h = jnp.dot(o, w_ref[...], preferred_element_type=jnp.float32)
        h_ref[...] = h
        s = jnp.sum(h * as_ref[...], axis=1, keepdims=True)
        d = jnp.sum(h * ad_ref[...], axis=1, keepdims=True)
        s_ref[...] = s
        d_ref[...] = d
        _g_update(i, s, d, g_ref, ms_ref, md_ref)

    out = pl.pallas_call(
        body,
        grid=(_NB,),
        in_specs=[
            pl.BlockSpec((_NC, _R, Cp), lambda i: (0, i, 0)),
            pl.BlockSpec((_NC, _R, 1), lambda i: (0, i, 0)),
            pl.BlockSpec((1, Cp), lambda i: (0, 0)),
            pl.BlockSpec((Cp, C), lambda i: (0, 0)),
            pl.BlockSpec((1, C), lambda i: (0, 0)),
            pl.BlockSpec((1, C), lambda i: (0, 0)),
            pl.BlockSpec((_R, Cp), lambda i: (i, 0)),
            pl.BlockSpec((_R, 1), lambda i: (i, 0)),
            pl.BlockSpec((_R, 1), lambda i: (i, 0)),
            pl.BlockSpec((1, 1), lambda i: (0, 0)),
        ],
        out_specs=[
            pl.BlockSpec((_R, C), lambda i: (i, 0)),
            pl.BlockSpec((_R, 1), lambda i: (i, 0)),
            pl.BlockSpec((_R, 1), lambda i: (i, 0)),
            pl.BlockSpec((1, 1), lambda i: (0, 0)),
            pl.BlockSpec((1, 1), lambda i: (0, 0)),
            pl.BlockSpec((1, 1), lambda i: (0, 0)),
        ],
        out_shape=[
            jax.ShapeDtypeStruct((_NPAD, C), jnp.float32),
            jax.ShapeDtypeStruct((_NPAD, 1), jnp.float32),
            jax.ShapeDtypeStruct((_NPAD, 1), jnp.float32),
            jax.ShapeDtypeStruct((1, 1), jnp.float32),
            jax.ShapeDtypeStruct((1, 1), jnp.float32),
            jax.ShapeDtypeStruct((1, 1), jnp.float32),
        ],
    )(acc, den, b, W, a_s, a_d, hp, sp, dp, gp)
    return out[0], out[1], out[2], out[3]


def _tc_fin(acc, den, b3, Wl3, bl3, x_pad, Wl2, bl2, hp, sp, dp, gp):
    def body(acc_ref, den_ref, b3_ref, wl3_ref, bl3_ref,
             x_ref, wl2_ref, bl2_ref, hp_ref, sp_ref, dp_ref, gp_ref,
             o_ref):
        sd = sp_ref[...] + dp_ref[...]
        esel = jnp.exp(jnp.where(sd >= 0.0, sd, 0.2 * sd) - gp_ref[...])
        a = acc_ref[0] + acc_ref[1] + esel * hp_ref[...]
        dn = den_ref[0] + den_ref[1] + esel
        h3 = jnp.where(dn > 0.0, a / dn, 0.0) + b3_ref[...]
        x2 = jnp.dot(h3, wl3_ref[...],
                     preferred_element_type=jnp.float32) + bl3_ref[...]
        x1 = jnp.dot(x_ref[...], wl2_ref[...],
                     preferred_element_type=jnp.float32) + bl2_ref[...]
        o_ref[...] = x1 + jnp.maximum(x2, 0.0)

    return pl.pallas_call(
        body,
        grid=(_NB,),
        in_specs=[
            pl.BlockSpec((_NC, _R, 128), lambda i: (0, i, 0)),
            pl.BlockSpec((_NC, _R, 1), lambda i: (0, i, 0)),
            pl.BlockSpec((1, 128), lambda i: (0, 0)),
            pl.BlockSpec((128, 128), lambda i: (0, 0)),
            pl.BlockSpec((1, 128), lambda i: (0, 0)),
            pl.BlockSpec((_R, 128), lambda i: (i, 0)),
            pl.BlockSpec((128, 128), lambda i: (0, 0)),
            pl.BlockSpec((1, 128), lambda i: (0, 0)),
            pl.BlockSpec((_R, 128), lambda i: (i, 0)),
            pl.BlockSpec((_R, 1), lambda i: (i, 0)),
            pl.BlockSpec((_R, 1), lambda i: (i, 0)),
            pl.BlockSpec((1, 1), lambda i: (0, 0)),
        ],
        out_specs=pl.BlockSpec((_R, 128), lambda i: (i, 0)),
        out_shape=jax.ShapeDtypeStruct((_NPAD, 128), jnp.float32),
    )(acc, den, b3, Wl3, bl3, x_pad, Wl2, bl2, hp, sp, dp, gp)


def kernel(x, edge_index, W1, a_src1, a_dst1, b1, W2, a_src2, a_dst2, b2,
           W3, a_src3, a_dst3, b3, Wl2, bl2, Wl3, bl3):
    idt = edge_index.dtype

    def pack_edges(K):
        ep = _epad(K)
        padn = jnp.full((ep - _E,), _N, idt)
        src = jnp.concatenate([edge_index[0], padn])
        dst = jnp.concatenate([edge_index[1], padn])
        nch = ep // _NW // K
        return jnp.stack([src.reshape(_NW, nch, K),
                          dst.reshape(_NW, nch, K)], axis=2)

    edges64 = pack_edges(_K64)
    edges128 = pack_edges(_K128)

    x_pad = jnp.pad(x, ((0, _NPAD - _N), (0, 0)))
    z1 = jnp.zeros((_NPAD,), jnp.float32)
    z64 = jnp.zeros((_NPAD, 64), jnp.float32)
    z128 = jnp.zeros((_NPAD, 128), jnp.float32)

    r2 = lambda v: v.reshape(1, -1)
    g16 = lambda g: jnp.broadcast_to(g.reshape(1), (16,))

    h1, s1, d1, g1 = _tc_pre(x_pad, W1, r2(a_src1), r2(a_dst1))
    acc1, den1 = _sc_edge64(edges64, h1, s1.reshape(-1), d1.reshape(-1),
                            g16(g1), z64, z1)
    h2, s2, d2, g2 = _tc_mid(acc1, den1.reshape(_NC, _NPAD, 1), r2(b1),
                             W2, r2(a_src2), r2(a_dst2), h1, s1, d1, g1)
    acc2, den2 = _sc_edge64(edges64, h2, s2.reshape(-1), d2.reshape(-1),
                            g16(g2), z64, z1)
    h3, s3, d3, g3 = _tc_mid(acc2, den2.reshape(_NC, _NPAD, 1), r2(b2),
                             W3, r2(a_src3), r2(a_dst3), h2, s2, d2, g2)
    acc3, den3 = _sc_edge128(edges128, h3, s3.reshape(-1), d3.reshape(-1),
                             g16(g3), z128, z1)
    out = _tc_fin(acc3, den3.reshape(_NC, _NPAD, 1), r2(b3), Wl3, r2(bl3),
                  x_pad, Wl2, r2(bl2), h3, s3, d3, g3)
    return out[:_N]


# speedup vs baseline: 1.0042x; 1.0042x over previous
import functools

import jax
import jax.numpy as jnp
from jax import lax
from jax.experimental import pallas as pl
from jax.experimental.pallas import tpu as pltpu
from jax.experimental.pallas import tpu_sc as plsc

_N = 10000
_E = 320000
_NPAD = 10240
_NC, _NS = 2, 16
_NW = _NC * _NS
_RPT = _NPAD // _NS


def _epad(K):
    return ((_E + 2 * _NW * K - 1) // (2 * _NW * K)) * (2 * _NW * K)


def _make_sc_edge(C, K):
    mesh = plsc.VectorSubcoreMesh(core_axis_name="c", subcore_axis_name="s")
    T = _epad(K) // _NW
    NCH = T // K

    @functools.partial(
        pl.kernel,
        out_type=[
            jax.ShapeDtypeStruct((_NC, _NPAD, C), jnp.float32),
            jax.ShapeDtypeStruct((_NC, _NPAD), jnp.float32),
        ],
        mesh=mesh,
        compiler_params=pltpu.CompilerParams(needs_layout_passes=False,
                                             use_tc_tiling_on_sc=False),
        scratch_types=[
            pltpu.VMEM((_NPAD,), jnp.float32),
            pltpu.VMEM((_NPAD,), jnp.float32),
            pltpu.VMEM((16,), jnp.float32),
            pltpu.VMEM((2, 2, K), jnp.int32),
            pltpu.VMEM((2, K, C), jnp.float32),
            pltpu.VMEM((2, K), jnp.float32),
            pltpu.VMEM_SHARED((_NPAD, C), jnp.float32),
            pltpu.VMEM_SHARED((_NPAD,), jnp.float32),
            pltpu.SemaphoreType.DMA,
            pltpu.SemaphoreType.DMA,
            pltpu.SemaphoreType.DMA,
            pltpu.SemaphoreType.DMA,
            pltpu.SemaphoreType.DMA,
            pltpu.SemaphoreType.DMA,
        ],
    )
    def sc_edge(edges_hbm, h_hbm, asrc_hbm, adst_hbm, g_hbm,
                z2_hbm, z1_hbm, acc_out, den_out,
                asrc_l, adst_l, g_l, ebuf, rows, evals, acc_sh, den_sh,
                isem0, isem1, gsem0, gsem1, ssem0, ssem1):
        cid = lax.axis_index("c")
        sid = lax.axis_index("s")
        wid = sid * _NC + cid
        r0 = sid * _RPT
        isem = (isem0, isem1)
        gsem = (gsem0, gsem1)
        ssem = (ssem0, ssem1)

        pltpu.sync_copy(z2_hbm.at[pl.ds(r0, _RPT)], acc_sh.at[pl.ds(r0, _RPT)])
        pltpu.sync_copy(z1_hbm.at[pl.ds(r0, _RPT)], den_sh.at[pl.ds(r0, _RPT)])
        pltpu.sync_copy(asrc_hbm, asrc_l)
        pltpu.sync_copy(adst_hbm, adst_l)
        pltpu.sync_copy(g_hbm, g_l)
        plsc.subcore_barrier()

        g = g_l[...]

        def idx_start(ci, b):
            pltpu.async_copy(edges_hbm.at[wid, ci], ebuf.at[b], isem[b])

        def idx_wait(b):
            pltpu.make_async_copy(edges_hbm.at[wid, 0], ebuf.at[b],
                                  isem[b]).wait()

        def gather_start(b):
            pltpu.async_copy(h_hbm.at[ebuf.at[b, 0]], rows.at[b], gsem[b])

        def gather_wait(b):
            pltpu.make_async_copy(h_hbm.at[pl.ds(0, K)], rows.at[b],
                                  gsem[b]).wait()

        def scatter_start(b):
            pltpu.async_copy(rows.at[b], acc_sh.at[ebuf.at[b, 1]], ssem[b],
                             add=True)
            pltpu.async_copy(evals.at[b], den_sh.at[ebuf.at[b, 1]], ssem[b],
                             add=True)

        def scatter_wait(b):
            pltpu.make_async_copy(z2_hbm.at[pl.ds(0, K)], rows.at[b],
                                  ssem[b]).wait()
            pltpu.make_async_copy(z1_hbm.at[pl.ds(0, K)], evals.at[b],
                                  ssem[b]).wait()

        idx_start(0, 0)
        idx_wait(0)
        gather_start(0)

        def pair(gi, _):
            for b in (0, 1):
                ci = 2 * gi + b
                o = 1 - b

                @pl.when(ci >= 1)
                def _():
                    scatter_wait(o)

                @pl.when(ci + 1 < NCH)
                def _():
                    idx_start(ci + 1, o)
                    idx_wait(o)
                    gather_start(o)

                def ebody(i, _):
                    sv = ebuf[b, 0, pl.ds(i * 16, 16)]
                    dv = ebuf[b, 1, pl.ds(i * 16, 16)]
                    s = (plsc.load_gather(asrc_l, [sv])
                         + plsc.load_gather(adst_l, [dv]))
                    alpha = jnp.where(s >= 0.0, s, 0.2 * s)
                    evals[b, pl.ds(i * 16, 16)] = jnp.exp(alpha - g)
                    return 0

                lax.fori_loop(0, K // 16, ebody, 0)

                gather_wait(b)

                def sbody(i, _):
                    e = evals[b, pl.ds(i * 16, 16)]
                    for jj in range(16):
                        es = e[jj]
                        j = i * 16 + jj
                        for r in range(C // 16):
                            rows[b, j, pl.ds(r * 16, 16)] = (
                                rows[b, j, pl.ds(r * 16, 16)] * es)
                    return 0

                lax.fori_loop(0, K // 16, sbody, 0)

                scatter_start(b)
            return 0

        lax.fori_loop(0, NCH // 2, pair, 0)
        scatter_wait(1)
        plsc.subcore_barrier()

        pltpu.sync_copy(acc_sh.at[pl.ds(r0, _RPT)],
                        acc_out.at[cid, pl.ds(r0, _RPT)])
        pltpu.sync_copy(den_sh.at[pl.ds(r0, _RPT)],
                        den_out.at[cid, pl.ds(r0, _RPT)])

    return sc_edge


_K64, _K128 = 128, 96
_sc_edge64 = _make_sc_edge(64, _K64)
_sc_edge128 = _make_sc_edge(128, _K128)


_R = 2560
_NB = _NPAD // _R


def _g_update(i, s, d, g_ref, ms_ref, md_ref):
    bs = jnp.broadcast_to(jnp.max(s), (1, 1))
    bd = jnp.broadcast_to(jnp.max(d), (1, 1))

    @pl.when(i == 0)
    def _():
        ms_ref[...] = bs
        md_ref[...] = bd

    @pl.when(i > 0)
    def _():
        ms_ref[...] = jnp.maximum(ms_ref[...], bs)
        md_ref[...] = jnp.maximum(md_ref[...], bd)

    @pl.when(i == _NB - 1)
    def _():
        g0 = ms_ref[0, 0] + md_ref[0, 0]
        g_ref[...] = jnp.broadcast_to(jnp.where(g0 >= 0.0, g0, 0.2 * g0),
                                      (1, 1))


def _tc_pre(x_pad, W, a_s, a_d):
    Ci = x_pad.shape[1]
    C = W.shape[1]

    def body(x_ref, w_ref, as_ref, ad_ref,
             h_ref, s_ref, d_ref, g_ref, ms_ref, md_ref):
        i = pl.program_id(0)
        h = jnp.dot(x_ref[...], w_ref[...], preferred_element_type=jnp.float32)
        h_ref[...] = h
        s = jnp.sum(h * as_ref[...], axis=1, keepdims=True)
        d = jnp.sum(h * ad_ref[...], axis=1, keepdims=True)
        s_ref[...] = s
        d_ref[...] = d
        _g_update(i, s, d, g_ref, ms_ref, md_ref)

    out = pl.pallas_call(
        body,
        grid=(_NB,),
        in_specs=[
            pl.BlockSpec((_R, Ci), lambda i: (i, 0)),
            pl.BlockSpec((Ci, C), lambda i: (0, 0)),
            pl.BlockSpec((1, C), lambda i: (0, 0)),
            pl.BlockSpec((1, C), lambda i: (0, 0)),
        ],
        out_specs=[
            pl.BlockSpec((_R, C), lambda i: (i, 0)),
            pl.BlockSpec((_R, 1), lambda i: (i, 0)),
            pl.BlockSpec((_R, 1), lambda i: (i, 0)),
            pl.BlockSpec((1, 1), lambda i: (0, 0)),
            pl.BlockSpec((1, 1), lambda i: (0, 0)),
            pl.BlockSpec((1, 1), lambda i: (0, 0)),
        ],
        out_shape=[
            jax.ShapeDtypeStruct((_NPAD, C), jnp.float32),
            jax.ShapeDtypeStruct((_NPAD, 1), jnp.float32),
            jax.ShapeDtypeStruct((_NPAD, 1), jnp.float32),
            jax.ShapeDtypeStruct((1, 1), jnp.float32),
            jax.ShapeDtypeStruct((1, 1), jnp.float32),
            jax.ShapeDtypeStruct((1, 1), jnp.float32),
        ],
    )(x_pad, W, a_s, a_d)
    return out[0], out[1], out[2], out[3]


def _tc_mid(acc, den, b, W, a_s, a_d, hp, sp, dp, gp):
    C = W.shape[1]

    Cp = hp.shape[1]

    def body(acc_ref, den_ref, b_ref, w_ref, as_ref, ad_ref,
             hp_ref, sp_ref, dp_ref, gp_ref,
             h_ref, s_ref, d_ref, g_ref, ms_ref, md_ref):
        i = pl.program_id(0)
        sd = sp_ref[...] + dp_ref[...]
        esel = jnp.exp(jnp.where(sd >= 0.0, sd, 0.2 * sd) - gp_ref[...])
        a = acc_ref[0] + acc_ref[1] + esel * hp_ref[...]
        dn = den_ref[0] + den_ref[1] + esel
        o = jnp.where(dn > 0.0, a / dn, 0.0) + b_ref[...]
        o = jnp.maximum(o, 0.0)
        h = jnp.dot(o, w_ref[...], preferred_element_type=jnp.float32)
        h_ref[...] = h
        s = jnp.sum(h * as_ref[...], axis=1, keepdims=True)
        d = jnp.sum(h * ad_ref[...], axis=1, keepdims=True)
        s_ref[...] = s
        d_ref[...] = d
        _g_update(i, s, d, g_ref, ms_ref, md_ref)

    out = pl.pallas_call(
        body,
        grid=(_NB,),
        in_specs=[
            pl.BlockSpec((_NC, _R, Cp), lambda i: (0, i, 0)),
            pl.BlockSpec((_NC, _R, 1), lambda i: (0, i, 0)),
            pl.BlockSpec((1, Cp), lambda i: (0, 0)),
            pl.BlockSpec((Cp, C), lambda i: (0, 0)),
            pl.BlockSpec((1, C), lambda i: (0, 0)),
            pl.BlockSpec((1, C), lambda i: (0, 0)),
            pl.BlockSpec((_R, Cp), lambda i: (i, 0)),
            pl.BlockSpec((_R, 1), lambda i: (i, 0)),
            pl.BlockSpec((_R, 1), lambda i: (i, 0)),
            pl.BlockSpec((1, 1), lambda i: (0, 0)),
        ],
        out_specs=[
            pl.BlockSpec((_R, C), lambda i: (i, 0)),
            pl.BlockSpec((_R, 1), lambda i: (i, 0)),
            pl.BlockSpec((_R, 1), lambda i: (i, 0)),
            pl.BlockSpec((1, 1), lambda i: (0, 0)),
            pl.BlockSpec((1, 1), lambda i: (0, 0)),
            pl.BlockSpec((1, 1), lambda i: (0, 0)),
        ],
        out_shape=[
            jax.ShapeDtypeStruct((_NPAD, C), jnp.float32),
            jax.ShapeDtypeStruct((_NPAD, 1), jnp.float32),
            jax.ShapeDtypeStruct((_NPAD, 1), jnp.float32),
            jax.ShapeDtypeStruct((1, 1), jnp.float32),
            jax.ShapeDtypeStruct((1, 1), jnp.float32),
            jax.ShapeDtypeStruct((1, 1), jnp.float32),
        ],
    )(acc, den, b, W, a_s, a_d, hp, sp, dp, gp)
    return out[0], out[1], out[2], out[3]


def _tc_fin(acc, den, b3, Wl3, bl3, x_pad, Wl2, bl2, hp, sp, dp, gp):
    def body(acc_ref, den_ref, b3_ref, wl3_ref, bl3_ref,
             x_ref, wl2_ref, bl2_ref, hp_ref, sp_ref, dp_ref, gp_ref,
             o_ref):
        sd = sp_ref[...] + dp_ref[...]
        esel = jnp.exp(jnp.where(sd >= 0.0, sd, 0.2 * sd) - gp_ref[...])
        a = acc_ref[0] + acc_ref[1] + esel * hp_ref[...]
        dn = den_ref[0] + den_ref[1] + esel
        h3 = jnp.where(dn > 0.0, a / dn, 0.0) + b3_ref[...]
        x2 = jnp.dot(h3, wl3_ref[...],
                     preferred_element_type=jnp.float32) + bl3_ref[...]
        x1 = jnp.dot(x_ref[...], wl2_ref[...],
                     preferred_element_type=jnp.float32) + bl2_ref[...]
        o_ref[...] = x1 + jnp.maximum(x2, 0.0)

    return pl.pallas_call(
        body,
        grid=(_NB,),
        in_specs=[
            pl.BlockSpec((_NC, _R, 128), lambda i: (0, i, 0)),
            pl.BlockSpec((_NC, _R, 1), lambda i: (0, i, 0)),
            pl.BlockSpec((1, 128), lambda i: (0, 0)),
            pl.BlockSpec((128, 128), lambda i: (0, 0)),
            pl.BlockSpec((1, 128), lambda i: (0, 0)),
            pl.BlockSpec((_R, 128), lambda i: (i, 0)),
            pl.BlockSpec((128, 128), lambda i: (0, 0)),
            pl.BlockSpec((1, 128), lambda i: (0, 0)),
            pl.BlockSpec((_R, 128), lambda i: (i, 0)),
            pl.BlockSpec((_R, 1), lambda i: (i, 0)),
            pl.BlockSpec((_R, 1), lambda i: (i, 0)),
            pl.BlockSpec((1, 1), lambda i: (0, 0)),
        ],
        out_specs=pl.BlockSpec((_R, 128), lambda i: (i, 0)),
        out_shape=jax.ShapeDtypeStruct((_NPAD, 128), jnp.float32),
    )(acc, den, b3, Wl3, bl3, x_pad, Wl2, bl2, hp, sp, dp, gp)


def kernel(x, edge_index, W1, a_src1, a_dst1, b1, W2, a_src2, a_dst2, b2,
           W3, a_src3, a_dst3, b3, Wl2, bl2, Wl3, bl3):
    idt = edge_index.dtype

    def pack_edges(K):
        ep = _epad(K)
        padn = jnp.full((ep - _E,), _N, idt)
        src = jnp.concatenate([edge_index[0], padn])
        dst = jnp.concatenate([edge_index[1], padn])
        nch = ep // _NW // K
        return jnp.stack([src.reshape(_NW, nch, K),
                          dst.reshape(_NW, nch, K)], axis=2)

    edges64 = pack_edges(_K64)
    edges128 = pack_edges(_K128)

    x_pad = jnp.pad(x, ((0, _NPAD - _N), (0, 0)))
    z1 = jnp.zeros((_NPAD,), jnp.float32)
    z64 = jnp.zeros((_NPAD, 64), jnp.float32)
    z128 = jnp.zeros((_NPAD, 128), jnp.float32)

    r2 = lambda v: v.reshape(1, -1)
    g16 = lambda g: jnp.broadcast_to(g.reshape(1), (16,))

    h1, s1, d1, g1 = _tc_pre(x_pad, W1, r2(a_src1), r2(a_dst1))
    acc1, den1 = _sc_edge64(edges64, h1, s1.reshape(-1), d1.reshape(-1),
                            g16(g1), z64, z1)
    h2, s2, d2, g2 = _tc_mid(acc1, den1.reshape(_NC, _NPAD, 1), r2(b1),
                             W2, r2(a_src2), r2(a_dst2), h1, s1, d1, g1)
    acc2, den2 = _sc_edge64(edges64, h2, s2.reshape(-1), d2.reshape(-1),
                            g16(g2), z64, z1)
    h3, s3, d3, g3 = _tc_mid(acc2, den2.reshape(_NC, _NPAD, 1), r2(b2),
                             W3, r2(a_src3), r2(a_dst3), h2, s2, d2, g2)
    acc3, den3 = _sc_edge128(edges128, h3, s3.reshape(-1), d3.reshape(-1),
                             g16(g3), z128, z1)
    out = _tc_fin(acc3, den3.reshape(_NC, _NPAD, 1), r2(b3), Wl3, r2(bl3),
                  x_pad, Wl2, r2(bl2), h3, s3, d3, g3)
    return out[:_N]
